# Initial kernel scaffold; baseline (speedup 1.0000x reference)
import jax
import jax.numpy as jnp
from jax.experimental import pallas as pl


def kernel(q, k, v, rand_attn):
    raise NotImplementedError("write your pallas kernel here")


# speedup vs baseline: 1.9060x; 1.9060x over previous
import numpy as np
import jax
import jax.numpy as jnp
from jax.experimental import pallas as pl
from jax.experimental.pallas import tpu as pltpu

_B, _H, _S, _DH, _BLK = 2, 16, 4096, 64, 64
_NB = _S // _BLK
_R = 3
_M = _NB - 2
_SCALE = 1.0 / np.sqrt(_DH)


def _bigbird_kernel(rand_ref, q_ref, k_ref, v_ref, o_ref):
    h = pl.program_id(1)
    k_all = k_ref[0, 0]
    v_all = v_ref[0, 0]

    qg = jnp.concatenate(
        [q_ref[0, 0, 0:_BLK], q_ref[0, 0, _S - _BLK:_S]], axis=0
    )
    s = jax.lax.dot_general(
        qg, k_all, (((1,), (1,)), ((), ())),
        preferred_element_type=jnp.float32,
    ) * _SCALE
    s = s - jnp.max(s, axis=-1, keepdims=True)
    p = jnp.exp(s)
    p = p / jnp.sum(p, axis=-1, keepdims=True)
    og = jnp.dot(p, v_all, preferred_element_type=jnp.float32)
    o_ref[0, 0, 0:_BLK] = og[0:_BLK]
    o_ref[0, 0, _S - _BLK:_S] = og[_BLK:]

    def body(m, carry):
        qm = q_ref[0, 0, pl.ds((m + 1) * _BLK, _BLK)]
        r0 = rand_ref[h, m, 0]
        r1 = rand_ref[h, m, 1]
        r2 = rand_ref[h, m, 2]
        kg = jnp.concatenate(
            [
                k_ref[0, 0, 0:_BLK],
                k_ref[0, 0, _S - _BLK:_S],
                k_ref[0, 0, pl.ds(m * _BLK, 3 * _BLK)],
                k_ref[0, 0, pl.ds(r0 * _BLK, _BLK)],
                k_ref[0, 0, pl.ds(r1 * _BLK, _BLK)],
                k_ref[0, 0, pl.ds(r2 * _BLK, _BLK)],
            ],
            axis=0,
        )
        vg = jnp.concatenate(
            [
                v_ref[0, 0, 0:_BLK],
                v_ref[0, 0, _S - _BLK:_S],
                v_ref[0, 0, pl.ds(m * _BLK, 3 * _BLK)],
                v_ref[0, 0, pl.ds(r0 * _BLK, _BLK)],
                v_ref[0, 0, pl.ds(r1 * _BLK, _BLK)],
                v_ref[0, 0, pl.ds(r2 * _BLK, _BLK)],
            ],
            axis=0,
        )
        sm = jax.lax.dot_general(
            qm, kg, (((1,), (1,)), ((), ())),
            preferred_element_type=jnp.float32,
        ) * _SCALE
        sm = sm - jnp.max(sm, axis=-1, keepdims=True)
        pm = jnp.exp(sm)
        pm = pm / jnp.sum(pm, axis=-1, keepdims=True)
        om = jnp.dot(pm, vg, preferred_element_type=jnp.float32)
        o_ref[0, 0, pl.ds((m + 1) * _BLK, _BLK)] = om
        return carry

    jax.lax.fori_loop(0, _M, body, 0)


def kernel(q, k, v, rand_attn):
    rand = rand_attn.astype(jnp.int32)

    def _spec(b, h, rand_ref):
        return (b, h, 0, 0)

    qkv_spec = pl.BlockSpec((1, 1, _S, _DH), _spec)
    out = pl.pallas_call(
        _bigbird_kernel,
        grid_spec=pltpu.PrefetchScalarGridSpec(
            num_scalar_prefetch=1,
            grid=(_B, _H),
            in_specs=[qkv_spec, qkv_spec, qkv_spec],
            out_specs=qkv_spec,
        ),
        out_shape=jax.ShapeDtypeStruct((_B, _H, _S, _DH), jnp.float32),
        compiler_params=pltpu.CompilerParams(
            dimension_semantics=("parallel", "parallel"),
        ),
    )(rand, q, k, v)
    return out


# speedup vs baseline: 2.2733x; 1.1927x over previous
import numpy as np
import jax
import jax.numpy as jnp
from jax.experimental import pallas as pl
from jax.experimental.pallas import tpu as pltpu

_B, _H, _S, _DH, _BLK = 2, 16, 4096, 64, 64
_NB = _S // _BLK
_R = 3
_M = _NB - 2
_SCALE = 1.0 / np.sqrt(_DH)


def _bigbird_kernel(rand_ref, q_ref, k_ref, v_ref, o_ref):
    h = pl.program_id(1)
    k_all = k_ref[0, 0]
    v_all = v_ref[0, 0]

    qg = jnp.concatenate(
        [q_ref[0, 0, 0:_BLK], q_ref[0, 0, _S - _BLK:_S]], axis=0
    ) * _SCALE
    s = jax.lax.dot_general(
        qg, k_all, (((1,), (1,)), ((), ())),
        preferred_element_type=jnp.float32,
    )
    s = s - jnp.max(s, axis=-1, keepdims=True)
    p = jnp.exp(s)
    p = p / jnp.sum(p, axis=-1, keepdims=True)
    og = jnp.dot(p, v_all, preferred_element_type=jnp.float32)
    o_ref[0, 0, 0:_BLK] = og[0:_BLK]
    o_ref[0, 0, _S - _BLK:_S] = og[_BLK:]

    def one_row(m):
        qm = q_ref[0, 0, pl.ds((m + 1) * _BLK, _BLK)] * _SCALE
        r0 = rand_ref[h, m, 0]
        r1 = rand_ref[h, m, 1]
        r2 = rand_ref[h, m, 2]
        k_parts = [
            k_ref[0, 0, 0:_BLK],
            k_ref[0, 0, _S - _BLK:_S],
            k_ref[0, 0, pl.ds(m * _BLK, 3 * _BLK)],
            k_ref[0, 0, pl.ds(r0 * _BLK, _BLK)],
            k_ref[0, 0, pl.ds(r1 * _BLK, _BLK)],
            k_ref[0, 0, pl.ds(r2 * _BLK, _BLK)],
        ]
        v_parts = [
            v_ref[0, 0, 0:_BLK],
            v_ref[0, 0, _S - _BLK:_S],
            v_ref[0, 0, pl.ds(m * _BLK, 3 * _BLK)],
            v_ref[0, 0, pl.ds(r0 * _BLK, _BLK)],
            v_ref[0, 0, pl.ds(r1 * _BLK, _BLK)],
            v_ref[0, 0, pl.ds(r2 * _BLK, _BLK)],
        ]
        s_parts = [
            jax.lax.dot_general(
                qm, kp, (((1,), (1,)), ((), ())),
                preferred_element_type=jnp.float32,
            )
            for kp in k_parts
        ]
        mx = s_parts[0].max(axis=-1, keepdims=True)
        for sp in s_parts[1:]:
            mx = jnp.maximum(mx, sp.max(axis=-1, keepdims=True))
        e_parts = [jnp.exp(sp - mx) for sp in s_parts]
        denom = e_parts[0].sum(axis=-1, keepdims=True)
        for ep in e_parts[1:]:
            denom = denom + ep.sum(axis=-1, keepdims=True)
        acc = jnp.dot(e_parts[0], v_parts[0], preferred_element_type=jnp.float32)
        for ep, vp in zip(e_parts[1:], v_parts[1:]):
            acc = acc + jnp.dot(ep, vp, preferred_element_type=jnp.float32)
        o_ref[0, 0, pl.ds((m + 1) * _BLK, _BLK)] = acc / denom

    def body(i, carry):
        one_row(2 * i)
        one_row(2 * i + 1)
        return carry

    jax.lax.fori_loop(0, _M // 2, body, 0)


def kernel(q, k, v, rand_attn):
    rand = rand_attn.astype(jnp.int32)

    def _spec(b, h, rand_ref):
        return (b, h, 0, 0)

    qkv_spec = pl.BlockSpec((1, 1, _S, _DH), _spec)
    out = pl.pallas_call(
        _bigbird_kernel,
        grid_spec=pltpu.PrefetchScalarGridSpec(
            num_scalar_prefetch=1,
            grid=(_B, _H),
            in_specs=[qkv_spec, qkv_spec, qkv_spec],
            out_specs=qkv_spec,
        ),
        out_shape=jax.ShapeDtypeStruct((_B, _H, _S, _DH), jnp.float32),
        compiler_params=pltpu.CompilerParams(
            dimension_semantics=("parallel", "parallel"),
        ),
    )(rand, q, k, v)
    return out


# speedup vs baseline: 2.4547x; 1.0798x over previous
import numpy as np
import jax
import jax.numpy as jnp
from jax.experimental import pallas as pl
from jax.experimental.pallas import tpu as pltpu

_B, _H, _S, _DH, _BLK = 2, 16, 4096, 64, 64
_NB = _S // _BLK
_R = 3
_M = _NB - 2
_SCALE = 1.0 / np.sqrt(_DH)


def _bigbird_kernel(rand_ref, q_ref, k_ref, v_ref, o_ref,
                    qb_ref, kb_ref, vb_ref, kg_ref, vg_ref):
    h = pl.program_id(1)

    qb_ref[...] = (q_ref[0, 0] * _SCALE).astype(jnp.bfloat16)
    kb_ref[...] = k_ref[0, 0].astype(jnp.bfloat16)
    vb_ref[...] = v_ref[0, 0].astype(jnp.bfloat16)
    kg_ref[0:_BLK] = kb_ref[0:_BLK]
    kg_ref[_BLK:2 * _BLK] = kb_ref[_S - _BLK:_S]
    vg_ref[0:_BLK] = vb_ref[0:_BLK]
    vg_ref[_BLK:2 * _BLK] = vb_ref[_S - _BLK:_S]

    qg = jnp.concatenate(
        [qb_ref[0:_BLK], qb_ref[_S - _BLK:_S]], axis=0
    )
    k_all = kb_ref[...]
    v_all = vb_ref[...]
    s = jax.lax.dot_general(
        qg, k_all, (((1,), (1,)), ((), ())),
        preferred_element_type=jnp.float32,
    )
    s = s - jnp.max(s, axis=-1, keepdims=True)
    p = jnp.exp(s)
    denom_g = jnp.sum(p, axis=-1, keepdims=True)
    og = jnp.dot(p.astype(jnp.bfloat16), v_all,
                 preferred_element_type=jnp.float32) / denom_g
    o_ref[0, 0, 0:_BLK] = og[0:_BLK]
    o_ref[0, 0, _S - _BLK:_S] = og[_BLK:]

    def one_row(m):
        qm = qb_ref[pl.ds((m + 1) * _BLK, _BLK)]
        r0 = rand_ref[h, m, 0]
        r1 = rand_ref[h, m, 1]
        r2 = rand_ref[h, m, 2]
        k_parts = [
            kg_ref[...],
            kb_ref[pl.ds(m * _BLK, 3 * _BLK)],
            kb_ref[pl.ds(r0 * _BLK, _BLK)],
            kb_ref[pl.ds(r1 * _BLK, _BLK)],
            kb_ref[pl.ds(r2 * _BLK, _BLK)],
        ]
        v_parts = [
            vg_ref[...],
            vb_ref[pl.ds(m * _BLK, 3 * _BLK)],
            vb_ref[pl.ds(r0 * _BLK, _BLK)],
            vb_ref[pl.ds(r1 * _BLK, _BLK)],
            vb_ref[pl.ds(r2 * _BLK, _BLK)],
        ]
        s_parts = [
            jax.lax.dot_general(
                qm, kp, (((1,), (1,)), ((), ())),
                preferred_element_type=jnp.float32,
            )
            for kp in k_parts
        ]
        mx = s_parts[0].max(axis=-1, keepdims=True)
        for sp in s_parts[1:]:
            mx = jnp.maximum(mx, sp.max(axis=-1, keepdims=True))
        e_parts = [jnp.exp(sp - mx) for sp in s_parts]
        denom = e_parts[0].sum(axis=-1, keepdims=True)
        for ep in e_parts[1:]:
            denom = denom + ep.sum(axis=-1, keepdims=True)
        acc = jnp.dot(e_parts[0].astype(jnp.bfloat16), v_parts[0],
                      preferred_element_type=jnp.float32)
        for ep, vp in zip(e_parts[1:], v_parts[1:]):
            acc = acc + jnp.dot(ep.astype(jnp.bfloat16), vp,
                                preferred_element_type=jnp.float32)
        o_ref[0, 0, pl.ds((m + 1) * _BLK, _BLK)] = acc / denom

    def body(i, carry):
        one_row(4 * i)
        one_row(4 * i + 1)
        one_row(4 * i + 2)
        one_row(4 * i + 3)
        return carry

    jax.lax.fori_loop(0, _M // 4, body, 0)
    one_row(_M - 2)
    one_row(_M - 1)


def kernel(q, k, v, rand_attn):
    rand = rand_attn.astype(jnp.int32)

    def _spec(b, h, rand_ref):
        return (b, h, 0, 0)

    qkv_spec = pl.BlockSpec((1, 1, _S, _DH), _spec)
    out = pl.pallas_call(
        _bigbird_kernel,
        grid_spec=pltpu.PrefetchScalarGridSpec(
            num_scalar_prefetch=1,
            grid=(_B, _H),
            in_specs=[qkv_spec, qkv_spec, qkv_spec],
            out_specs=qkv_spec,
            scratch_shapes=[
                pltpu.VMEM((_S, _DH), jnp.bfloat16),
                pltpu.VMEM((_S, _DH), jnp.bfloat16),
                pltpu.VMEM((_S, _DH), jnp.bfloat16),
                pltpu.VMEM((2 * _BLK, _DH), jnp.bfloat16),
                pltpu.VMEM((2 * _BLK, _DH), jnp.bfloat16),
            ],
        ),
        out_shape=jax.ShapeDtypeStruct((_B, _H, _S, _DH), jnp.float32),
        compiler_params=pltpu.CompilerParams(
            dimension_semantics=("parallel", "parallel"),
        ),
    )(rand, q, k, v)
    return out
